# baseline (speedup 1.0000x reference)
import dataclasses
import functools

import jax
import jax.numpy as jnp
from jax.experimental import pallas as pl
from jax.experimental.pallas import tpu as pltpu
from jax.experimental.pallas import tpu_sc as plsc

REDUCTION_FACTOR = 1


NSTREAM = 4
NH_SC = 8
N_UNITS = 32


def _scores_body(*args, num_tc_heads):
    x_refs = args[:NSTREAM]
    (s_ref,) = args[NSTREAM:]
    g = pl.program_id(0)
    l = pl.program_id(1)
    q = num_tc_heads // NSTREAM
    for k, ref in enumerate(x_refs):
        x = ref[0]
        maxv = jnp.max(x, axis=1, keepdims=True)
        part = jnp.sum(maxv)
        idx = g + k * q

        @pl.when(l == 0)
        def _init(idx=idx, part=part):
            s_ref[idx] = part

        @pl.when(l > 0)
        def _add(idx=idx, part=part):
            s_ref[idx] += part


def _sc_scores_body(
    a_hbm, out_hbm, buf_ref, sacc_ref, *, head0, rows_per_unit, t_size, rb
):
    core = jax.lax.axis_index("core")
    sid = jax.lax.axis_index("subcore")
    u = core * 16 + sid
    uph = N_UNITS // NH_SC
    hl = u // uph
    r_base = (u % uph) * rows_per_unit
    sacc_ref[...] = jnp.zeros((16,), jnp.float32)

    @pl.loop(0, rows_per_unit, step=rb)
    def _chunk(r):
        pltpu.sync_copy(a_hbm.at[head0 + hl, pl.ds(r_base + r, rb)], buf_ref)
        for rr in range(rb):
            vs = [buf_ref[rr, pl.ds(16 * k, 16)] for k in range(t_size // 16)]
            while len(vs) > 1:
                nxt = [
                    jnp.maximum(vs[i], vs[i + 1])
                    for i in range(0, len(vs) - 1, 2)
                ]
                if len(vs) % 2:
                    nxt.append(vs[-1])
                vs = nxt
            sacc_ref[...] += jnp.max(vs[0])

    pltpu.sync_copy(sacc_ref, out_hbm.at[u])


def _select_body(tc_s_ref, sc_s_ref, head_ref, focus_ref, *, num_tc_heads, l_size):
    uph = N_UNITS // NH_SC

    def body(i, carry):
        m, idx = carry
        v = tc_s_ref[i]
        better = v > m
        return jnp.where(better, v, m), jnp.where(better, i, idx)

    m, idx = jax.lax.fori_loop(
        0, num_tc_heads, body, (jnp.float32(-jnp.inf), jnp.int32(0))
    )

    def body2(hl, carry):
        m, idx = carry
        s = sc_s_ref[hl * uph, 0]
        for j in range(1, uph):
            s = s + sc_s_ref[hl * uph + j, 0]
        better = s > m
        return (
            jnp.where(better, s, m),
            jnp.where(better, num_tc_heads + hl, idx),
        )

    m, idx = jax.lax.fori_loop(0, NH_SC, body2, (m, idx))
    head_ref[0] = idx
    focus_ref[0] = m / l_size


def _argmax_body(head_ref, x_ref, am_ref, *, t_size):
    del head_ref
    x = x_ref[0]
    maxv = jnp.max(x, axis=1, keepdims=True)
    ti = jax.lax.broadcasted_iota(jnp.int32, x.shape, 1)
    am_ref[...] = jnp.min(jnp.where(x == maxv, ti, t_size), axis=1, keepdims=True)


def _hist_body(
    am_hbm, out_hbm, idx_ref, hist_ref, ident_ref, shared_ref, *, l_size, t_size
):
    core = jax.lax.axis_index("core")
    sid = jax.lax.axis_index("subcore")
    n_sub = 16
    per_sub = l_size // n_sub
    n_vec = t_size // 16

    @pl.when(core == 0)
    def _():
        @pl.loop(0, n_vec)
        def _zero(i):
            hist_ref[pl.ds(i * 16, 16)] = jnp.zeros((16,), jnp.int32)
            ident_ref[pl.ds(i * 16, 16)] = jax.lax.iota(jnp.int32, 16) + i * 16

        @pl.when(sid == 0)
        def _init_shared():
            pltpu.sync_copy(hist_ref, shared_ref)

        plsc.subcore_barrier()

        pltpu.sync_copy(am_hbm.at[pl.ds(sid * per_sub, per_sub)], idx_ref)
        @pl.loop(0, per_sub, step=16)
        def _bin(k):
            v = idx_ref[pl.ds(k, 16)]
            counts, last = plsc.scan_count(v)
            plsc.addupdate_scatter(hist_ref, [v], counts, mask=last)

        pltpu.sync_copy(hist_ref, shared_ref.at[ident_ref], add=True)
        plsc.subcore_barrier()

        @pl.when(sid == 0)
        def _out():
            pltpu.sync_copy(shared_ref, out_hbm)


def kernel(att_ws):
    L = att_ws.shape[-2]
    T = att_ws.shape[-1]
    a = jnp.reshape(att_ws, (-1, L, T))
    H = a.shape[0]

    H_TC = H - NH_SC
    q = H_TC // NSTREAM
    LB = L // 2
    vector_mesh = plsc.VectorSubcoreMesh(
        core_axis_name="core", subcore_axis_name="subcore", num_cores=2,
        num_subcores=16,
    )
    sc_params = pltpu.CompilerParams()
    if "needs_layout_passes" in pltpu.CompilerParams.__dataclass_fields__:
        sc_params = dataclasses.replace(sc_params, needs_layout_passes=False)

    RPU = L // (N_UNITS // NH_SC)
    sc_scores = pl.kernel(
        functools.partial(
            _sc_scores_body, head0=H_TC, rows_per_unit=RPU, t_size=T, rb=4
        ),
        out_type=jax.ShapeDtypeStruct((N_UNITS, 16), jnp.float32),
        mesh=vector_mesh,
        compiler_params=sc_params,
        scratch_types=[
            pltpu.VMEM((4, T), jnp.float32),
            pltpu.VMEM((16,), jnp.float32),
        ],
    )(a)

    tc_scores = pl.pallas_call(
        functools.partial(_scores_body, num_tc_heads=H_TC),
        grid=(q, 2),
        in_specs=[
            pl.BlockSpec(
                (1, LB, T), functools.partial(lambda k, h, l: (h + k * q, l, 0), k)
            )
            for k in range(NSTREAM)
        ],
        out_specs=pl.BlockSpec(memory_space=pltpu.SMEM),
        out_shape=jax.ShapeDtypeStruct((H_TC,), jnp.float32),
        compiler_params=pltpu.CompilerParams(
            dimension_semantics=("arbitrary", "arbitrary")
        ),
    )(*([a] * NSTREAM))

    head, focus = pl.pallas_call(
        functools.partial(_select_body, num_tc_heads=H_TC, l_size=L),
        in_specs=[
            pl.BlockSpec(memory_space=pltpu.SMEM),
            pl.BlockSpec(memory_space=pltpu.SMEM),
        ],
        out_specs=(
            pl.BlockSpec(memory_space=pltpu.SMEM),
            pl.BlockSpec(memory_space=pltpu.SMEM),
        ),
        out_shape=(
            jax.ShapeDtypeStruct((1,), jnp.int32),
            jax.ShapeDtypeStruct((1,), jnp.float32),
        ),
    )(tc_scores, sc_scores)

    CHUNK = 512
    NCH = L // CHUNK
    grid_spec = pltpu.PrefetchScalarGridSpec(
        num_scalar_prefetch=1,
        grid=(NCH,),
        in_specs=[pl.BlockSpec((1, CHUNK, T), lambda i, h: (h[0], i, 0))],
        out_specs=pl.BlockSpec((CHUNK, 1), lambda i, h: (i, 0)),
    )
    am = pl.pallas_call(
        functools.partial(_argmax_body, t_size=T),
        grid_spec=grid_spec,
        out_shape=jax.ShapeDtypeStruct((L, 1), jnp.int32),
    )(head, a)

    hist = pl.kernel(
        functools.partial(_hist_body, l_size=L, t_size=T),
        out_type=jax.ShapeDtypeStruct((T,), jnp.int32),
        mesh=vector_mesh,
        compiler_params=sc_params,
        scratch_types=[
            pltpu.VMEM((L // 16,), jnp.int32),
            pltpu.VMEM((T,), jnp.int32),
            pltpu.VMEM((T,), jnp.int32),
            pltpu.VMEM_SHARED((T,), jnp.int32),
        ],
    )(am[:, 0])

    durations = hist * REDUCTION_FACTOR
    return (durations, focus[0])


# speedup vs baseline: 1.0211x; 1.0211x over previous
import dataclasses
import functools

import jax
import jax.numpy as jnp
from jax.experimental import pallas as pl
from jax.experimental.pallas import tpu as pltpu
from jax.experimental.pallas import tpu_sc as plsc

REDUCTION_FACTOR = 1


NSTREAM = 4


def _scores_body(*args, num_heads, l_size):
    x_refs = args[:NSTREAM]
    head_ref, focus_ref, acc_ref = args[NSTREAM:]
    g = pl.program_id(0)
    l = pl.program_id(1)
    nl = pl.num_programs(1)
    q = num_heads // NSTREAM
    for k, ref in enumerate(x_refs):
        x = ref[0]
        maxv = jnp.max(x, axis=1, keepdims=True)
        part = jnp.sum(maxv)
        idx = g + k * q

        @pl.when(l == 0)
        def _init(idx=idx, part=part):
            acc_ref[idx] = part

        @pl.when(l > 0)
        def _add(idx=idx, part=part):
            acc_ref[idx] += part

    @pl.when((g == q - 1) & (l == nl - 1))
    def _select():
        def body(i, carry):
            m, idx = carry
            v = acc_ref[i]
            better = v > m
            return jnp.where(better, v, m), jnp.where(better, i, idx)

        m, idx = jax.lax.fori_loop(
            0, num_heads, body, (jnp.float32(-jnp.inf), jnp.int32(0))
        )
        head_ref[0] = idx
        focus_ref[0] = m / l_size


def _argmax_body(head_ref, x_ref, am_ref, *, t_size):
    del head_ref
    x = x_ref[0]
    maxv = jnp.max(x, axis=1, keepdims=True)
    ti = jax.lax.broadcasted_iota(jnp.int32, x.shape, 1)
    am_ref[...] = jnp.min(jnp.where(x == maxv, ti, t_size), axis=1)


def _hist_body(
    am_hbm, out_hbm, idx_ref, hist_ref, ident_ref, shared_ref, *, l_size, t_size
):
    core = jax.lax.axis_index("core")
    sid = jax.lax.axis_index("subcore")
    n_sub = 16
    per_sub = l_size // n_sub
    n_vec = t_size // 16

    @pl.when(core == 0)
    def _():
        @pl.loop(0, n_vec)
        def _zero(i):
            hist_ref[pl.ds(i * 16, 16)] = jnp.zeros((16,), jnp.int32)
            ident_ref[pl.ds(i * 16, 16)] = jax.lax.iota(jnp.int32, 16) + i * 16

        @pl.when(sid == 0)
        def _init_shared():
            pltpu.sync_copy(hist_ref, shared_ref)

        plsc.subcore_barrier()

        pltpu.sync_copy(am_hbm.at[pl.ds(sid * per_sub, per_sub)], idx_ref)
        @pl.loop(0, per_sub, step=16)
        def _bin(k):
            v = idx_ref[pl.ds(k, 16)]
            counts, last = plsc.scan_count(v)
            plsc.addupdate_scatter(hist_ref, [v], counts, mask=last)

        pltpu.sync_copy(hist_ref, shared_ref.at[ident_ref], add=True)
        plsc.subcore_barrier()

        @pl.when(sid == 0)
        def _out():
            pltpu.sync_copy(shared_ref, out_hbm)


def kernel(att_ws):
    L = att_ws.shape[-2]
    T = att_ws.shape[-1]
    a = jnp.reshape(att_ws, (-1, L, T))
    H = a.shape[0]

    q = H // NSTREAM
    LB = L // 2
    head, focus = pl.pallas_call(
        functools.partial(_scores_body, num_heads=H, l_size=L),
        grid=(q, 2),
        in_specs=[
            pl.BlockSpec(
                (1, LB, T), functools.partial(lambda k, h, l: (h + k * q, l, 0), k)
            )
            for k in range(NSTREAM)
        ],
        out_specs=(
            pl.BlockSpec(memory_space=pltpu.SMEM),
            pl.BlockSpec(memory_space=pltpu.SMEM),
        ),
        out_shape=(
            jax.ShapeDtypeStruct((1,), jnp.int32),
            jax.ShapeDtypeStruct((1,), jnp.float32),
        ),
        scratch_shapes=[pltpu.SMEM((H,), jnp.float32)],
        compiler_params=pltpu.CompilerParams(
            dimension_semantics=("arbitrary", "arbitrary")
        ),
    )(*([a] * NSTREAM))

    CHUNK = 512
    NCH = L // CHUNK
    grid_spec = pltpu.PrefetchScalarGridSpec(
        num_scalar_prefetch=1,
        grid=(NCH,),
        in_specs=[pl.BlockSpec((1, CHUNK, T), lambda i, h: (h[0], i, 0))],
        out_specs=pl.BlockSpec((CHUNK,), lambda i, h: (i,)),
    )
    am = pl.pallas_call(
        functools.partial(_argmax_body, t_size=T),
        grid_spec=grid_spec,
        out_shape=jax.ShapeDtypeStruct((L,), jnp.int32),
    )(head, a)

    vector_mesh = plsc.VectorSubcoreMesh(
        core_axis_name="core", subcore_axis_name="subcore", num_cores=2,
        num_subcores=16,
    )
    sc_params = pltpu.CompilerParams()
    if "needs_layout_passes" in pltpu.CompilerParams.__dataclass_fields__:
        sc_params = dataclasses.replace(sc_params, needs_layout_passes=False)
    hist = pl.kernel(
        functools.partial(_hist_body, l_size=L, t_size=T),
        out_type=jax.ShapeDtypeStruct((T,), jnp.int32),
        mesh=vector_mesh,
        compiler_params=sc_params,
        scratch_types=[
            pltpu.VMEM((L // 16,), jnp.int32),
            pltpu.VMEM((T,), jnp.int32),
            pltpu.VMEM((T,), jnp.int32),
            pltpu.VMEM_SHARED((T,), jnp.int32),
        ],
    )(am)

    durations = hist * REDUCTION_FACTOR
    return (durations, focus[0])


# speedup vs baseline: 1.0300x; 1.0087x over previous
import dataclasses
import functools

import jax
import jax.numpy as jnp
from jax.experimental import pallas as pl
from jax.experimental.pallas import tpu as pltpu
from jax.experimental.pallas import tpu_sc as plsc

REDUCTION_FACTOR = 1


NSTREAM = 4


def _scores_body(*args, num_heads, l_size):
    x_refs = args[:NSTREAM]
    head_ref, focus_ref, acc_ref = args[NSTREAM:]
    g = pl.program_id(0)
    l = pl.program_id(1)
    nl = pl.num_programs(1)
    q = num_heads // NSTREAM
    for k, ref in enumerate(x_refs):
        x = ref[0]
        maxv = jnp.max(x, axis=1, keepdims=True)
        part = jnp.sum(maxv)
        idx = g + k * q

        @pl.when(l == 0)
        def _init(idx=idx, part=part):
            acc_ref[idx] = part

        @pl.when(l > 0)
        def _add(idx=idx, part=part):
            acc_ref[idx] += part

    @pl.when((g == q - 1) & (l == nl - 1))
    def _select():
        def body(i, carry):
            m, idx = carry
            v = acc_ref[i]
            better = v > m
            return jnp.where(better, v, m), jnp.where(better, i, idx)

        m, idx = jax.lax.fori_loop(
            0, num_heads, body, (jnp.float32(-jnp.inf), jnp.int32(0))
        )
        head_ref[0] = idx
        focus_ref[0] = m / l_size


def _argmax_body(head_ref, x_ref, am_ref, *, t_size):
    del head_ref
    x = x_ref[0]
    maxv = jnp.max(x, axis=1, keepdims=True)
    ti = jax.lax.broadcasted_iota(jnp.int32, x.shape, 1)
    am_ref[...] = jnp.min(jnp.where(x == maxv, ti, t_size), axis=1)


def _hist_body(
    am_hbm, out_hbm, idx_ref, hist_ref, ident_ref, shared_ref, *, l_size, t_size
):
    core = jax.lax.axis_index("core")
    sid = jax.lax.axis_index("subcore")
    n_sub = 16
    per_sub = l_size // n_sub
    n_vec = t_size // 16

    @pl.when(core == 0)
    def _():
        @pl.loop(0, n_vec)
        def _zero(i):
            hist_ref[pl.ds(i * 16, 16)] = jnp.zeros((16,), jnp.int32)
            ident_ref[pl.ds(i * 16, 16)] = jax.lax.iota(jnp.int32, 16) + i * 16

        @pl.when(sid == 0)
        def _init_shared():
            pltpu.sync_copy(hist_ref, shared_ref)

        plsc.subcore_barrier()

        pltpu.sync_copy(am_hbm.at[pl.ds(sid * per_sub, per_sub)], idx_ref)
        @pl.loop(0, per_sub, step=16)
        def _bin(k):
            v = idx_ref[pl.ds(k, 16)]
            counts, last = plsc.scan_count(v)
            plsc.addupdate_scatter(hist_ref, [v], counts, mask=last)

        pltpu.sync_copy(hist_ref, shared_ref.at[ident_ref], add=True)
        plsc.subcore_barrier()

        @pl.when(sid == 0)
        def _out():
            pltpu.sync_copy(shared_ref, out_hbm)


def kernel(att_ws):
    L = att_ws.shape[-2]
    T = att_ws.shape[-1]
    a = jnp.reshape(att_ws, (-1, L, T))
    H = a.shape[0]

    q = H // NSTREAM
    LB = L // 2
    head, focus = pl.pallas_call(
        functools.partial(_scores_body, num_heads=H, l_size=L),
        grid=(q, 2),
        in_specs=[
            pl.BlockSpec(
                (1, LB, T), functools.partial(lambda k, h, l: (h + k * q, l, 0), k)
            )
            for k in range(NSTREAM)
        ],
        out_specs=(
            pl.BlockSpec(memory_space=pltpu.SMEM),
            pl.BlockSpec(memory_space=pltpu.SMEM),
        ),
        out_shape=(
            jax.ShapeDtypeStruct((1,), jnp.int32),
            jax.ShapeDtypeStruct((1,), jnp.float32),
        ),
        scratch_shapes=[pltpu.SMEM((H,), jnp.float32)],
        compiler_params=pltpu.CompilerParams(
            dimension_semantics=("arbitrary", "arbitrary")
        ),
    )(*([a] * NSTREAM))

    CHUNK = 1024
    NCH = L // CHUNK
    grid_spec = pltpu.PrefetchScalarGridSpec(
        num_scalar_prefetch=1,
        grid=(NCH,),
        in_specs=[pl.BlockSpec((1, CHUNK, T), lambda i, h: (h[0], i, 0))],
        out_specs=pl.BlockSpec((CHUNK,), lambda i, h: (i,)),
    )
    am = pl.pallas_call(
        functools.partial(_argmax_body, t_size=T),
        grid_spec=grid_spec,
        out_shape=jax.ShapeDtypeStruct((L,), jnp.int32),
    )(head, a)

    vector_mesh = plsc.VectorSubcoreMesh(
        core_axis_name="core", subcore_axis_name="subcore", num_cores=1,
        num_subcores=16,
    )
    sc_params = pltpu.CompilerParams()
    if "needs_layout_passes" in pltpu.CompilerParams.__dataclass_fields__:
        sc_params = dataclasses.replace(sc_params, needs_layout_passes=False)
    hist = pl.kernel(
        functools.partial(_hist_body, l_size=L, t_size=T),
        out_type=jax.ShapeDtypeStruct((T,), jnp.int32),
        mesh=vector_mesh,
        compiler_params=sc_params,
        scratch_types=[
            pltpu.VMEM((L // 16,), jnp.int32),
            pltpu.VMEM((T,), jnp.int32),
            pltpu.VMEM((T,), jnp.int32),
            pltpu.VMEM_SHARED((T,), jnp.int32),
        ],
    )(am)

    durations = hist * REDUCTION_FACTOR
    return (durations, focus[0])
